# baseline (device time: 18345 ns/iter reference)
import jax
import jax.numpy as jnp
from jax import lax
from jax.experimental import pallas as pl
from jax.experimental.pallas import tpu as pltpu

NC = 4


def kernel(x, dy):
    k, d = x.shape
    _, f = dy.shape
    half = d // 2
    q = half // 2
    qc = q // NC

    def body(x_ref, dy_ref, out_ref, send_ref, recv_ref,
             sem_ys, sem_yr, sem_xs, sem_xr):
        my_x = lax.axis_index("x")
        my_y = lax.axis_index("y")
        my_z = lax.axis_index("z")
        other = 1 - my_y
        y_nbr = (my_x, other, my_z)
        x_nbr = (1 - my_x, my_y, my_z)

        barrier_sem = pltpu.get_barrier_semaphore()
        for nbr in (y_nbr, x_nbr):
            pl.semaphore_signal(
                barrier_sem, inc=1, device_id=nbr,
                device_id_type=pl.DeviceIdType.MESH,
            )

        dyb = dy_ref[:, :].astype(jnp.bfloat16)

        x_sq = x_ref[:, pl.ds(other * half + my_x * q, q)].astype(jnp.bfloat16)
        part_sq = lax.dot_general(
            x_sq, dyb, (((0,), (0,)), ((), ())),
            preferred_element_type=jnp.float32,
        )
        send_ref[:, :] = part_sq.astype(jnp.bfloat16)

        pl.semaphore_wait(barrier_sem, 2)

        rdma_y = []
        for c in range(NC):
            r = pltpu.make_async_remote_copy(
                src_ref=send_ref.at[pl.ds(c * qc, qc)],
                dst_ref=recv_ref.at[pl.ds(my_x * q + c * qc, qc)],
                send_sem=sem_ys.at[c],
                recv_sem=sem_yr.at[c],
                device_id=y_nbr,
                device_id_type=pl.DeviceIdType.MESH,
            )
            r.start()
            rdma_y.append(r)

        x_keep = x_ref[:, pl.ds(my_y * half, half)].astype(jnp.bfloat16)
        part_keep = lax.dot_general(
            x_keep, dyb, (((0,), (0,)), ((), ())),
            preferred_element_type=jnp.float32,
        )
        out_ref[:, :] = part_keep

        rdma_x = []
        for c in range(NC):
            rdma_y[c].wait_recv()
            rc = pl.ds(my_x * q + c * qc, qc)
            r = pltpu.make_async_remote_copy(
                src_ref=recv_ref.at[rc],
                dst_ref=recv_ref.at[rc],
                send_sem=sem_xs.at[c],
                recv_sem=sem_xr.at[c],
                device_id=x_nbr,
                device_id_type=pl.DeviceIdType.MESH,
            )
            r.start()
            rdma_x.append(r)
            out_ref[rc, :] = out_ref[rc, :] + recv_ref[rc, :].astype(jnp.float32)

        for c in range(NC):
            rdma_x[c].wait_recv()
            rc = pl.ds((1 - my_x) * q + c * qc, qc)
            out_ref[rc, :] = out_ref[rc, :] + recv_ref[rc, :].astype(jnp.float32)

        for c in range(NC):
            rdma_y[c].wait_send()
            rdma_x[c].wait_send()

    return pl.pallas_call(
        body,
        out_shape=jax.ShapeDtypeStruct((half, f), jnp.float32),
        in_specs=[
            pl.BlockSpec(memory_space=pltpu.VMEM),
            pl.BlockSpec(memory_space=pltpu.VMEM),
        ],
        out_specs=pl.BlockSpec(memory_space=pltpu.VMEM),
        scratch_shapes=[
            pltpu.VMEM((q, f), jnp.bfloat16),
            pltpu.VMEM((half, f), jnp.bfloat16),
            pltpu.SemaphoreType.DMA((NC,)),
            pltpu.SemaphoreType.DMA((NC,)),
            pltpu.SemaphoreType.DMA((NC,)),
            pltpu.SemaphoreType.DMA((NC,)),
        ],
        compiler_params=pltpu.CompilerParams(collective_id=0),
    )(x, dy)


# device time: 18247 ns/iter; 1.0054x vs baseline; 1.0054x over previous
import jax
import jax.numpy as jnp
from jax import lax
from jax.experimental import pallas as pl
from jax.experimental.pallas import tpu as pltpu

NCF = 8


def kernel(x, dy):
    k, d = x.shape
    _, f = dy.shape
    half = d // 2
    q = half // 2
    fc = f // NCF

    def body(x_ref, dy_ref, out_ref, dyb_ref, send_ref, recv_ref,
             sem_ys, sem_yr, sem_xs, sem_xr):
        my_x = lax.axis_index("x")
        my_y = lax.axis_index("y")
        my_z = lax.axis_index("z")
        other = 1 - my_y
        y_nbr = (my_x, other, my_z)
        x_nbr = (1 - my_x, my_y, my_z)

        barrier_sem = pltpu.get_barrier_semaphore()
        for nbr in (y_nbr, x_nbr):
            pl.semaphore_signal(
                barrier_sem, inc=1, device_id=nbr,
                device_id_type=pl.DeviceIdType.MESH,
            )

        x_sq = x_ref[:, pl.ds(other * half + my_x * q, q)].astype(jnp.bfloat16)

        pl.semaphore_wait(barrier_sem, 2)

        rdma_y = []
        for c in range(NCF):
            cs = pl.ds(c * fc, fc)
            dyb_c = dy_ref[:, cs].astype(jnp.bfloat16)
            dyb_ref[:, cs] = dyb_c
            ps = lax.dot_general(
                x_sq, dyb_c, (((0,), (0,)), ((), ())),
                preferred_element_type=jnp.float32,
            )
            send_ref[c, :, :] = ps.astype(jnp.bfloat16)
            r = pltpu.make_async_remote_copy(
                src_ref=send_ref.at[c],
                dst_ref=recv_ref.at[0, c],
                send_sem=sem_ys.at[c],
                recv_sem=sem_yr.at[c],
                device_id=y_nbr,
                device_id_type=pl.DeviceIdType.MESH,
            )
            r.start()
            rdma_y.append(r)

        x_keep = x_ref[:, pl.ds(my_y * half, half)].astype(jnp.bfloat16)
        part_keep = lax.dot_general(
            x_keep, dyb_ref[:, :], (((0,), (0,)), ((), ())),
            preferred_element_type=jnp.float32,
        )
        out_ref[:, :] = part_keep

        yq = pl.ds(my_x * q, q)
        rdma_x = []
        for c in range(NCF):
            rdma_y[c].wait_recv()
            r = pltpu.make_async_remote_copy(
                src_ref=recv_ref.at[0, c],
                dst_ref=recv_ref.at[1, c],
                send_sem=sem_xs.at[c],
                recv_sem=sem_xr.at[c],
                device_id=x_nbr,
                device_id_type=pl.DeviceIdType.MESH,
            )
            r.start()
            rdma_x.append(r)
            cs = pl.ds(c * fc, fc)
            out_ref[yq, cs] = (
                out_ref[yq, cs] + recv_ref[0, c, :, :].astype(jnp.float32)
            )

        xq = pl.ds((1 - my_x) * q, q)
        for c in range(NCF):
            rdma_x[c].wait_recv()
            cs = pl.ds(c * fc, fc)
            out_ref[xq, cs] = (
                out_ref[xq, cs] + recv_ref[1, c, :, :].astype(jnp.float32)
            )

        for c in range(NCF):
            rdma_y[c].wait_send()
            rdma_x[c].wait_send()

    return pl.pallas_call(
        body,
        out_shape=jax.ShapeDtypeStruct((half, f), jnp.float32),
        in_specs=[
            pl.BlockSpec(memory_space=pltpu.VMEM),
            pl.BlockSpec(memory_space=pltpu.VMEM),
        ],
        out_specs=pl.BlockSpec(memory_space=pltpu.VMEM),
        scratch_shapes=[
            pltpu.VMEM((k, f), jnp.bfloat16),
            pltpu.VMEM((NCF, q, fc), jnp.bfloat16),
            pltpu.VMEM((2, NCF, q, fc), jnp.bfloat16),
            pltpu.SemaphoreType.DMA((NCF,)),
            pltpu.SemaphoreType.DMA((NCF,)),
            pltpu.SemaphoreType.DMA((NCF,)),
            pltpu.SemaphoreType.DMA((NCF,)),
        ],
        compiler_params=pltpu.CompilerParams(collective_id=0),
    )(x, dy)
